# baseline (speedup 1.0000x reference)
import jax
import jax.numpy as jnp
from jax.experimental import pallas as pl

HIDDEN_DIM = 1024
N_EXPERTS = 8
K_TOP = 2
TILE = 2048


def _router_kernel(x_ref, w_ref, scores_ref, wts_ref, idx_ref):
    x_tile = x_ref[...]
    w = w_ref[...]
    logits = jax.lax.dot_general(
        x_tile, w, (((1,), (1,)), ((), ())),
        preferred_element_type=jnp.float32,
    )
    scores = jax.nn.softplus(logits)
    norm = jnp.sum(scores, axis=1, keepdims=True)
    scores = scores / jnp.maximum(norm, 1e-12)

    col = jax.lax.broadcasted_iota(jnp.int32, scores.shape, 1)
    i1 = jnp.argmax(scores, axis=1)
    m1 = jnp.max(scores, axis=1)
    masked = jnp.where(col == i1[:, None], -jnp.inf, scores)
    i2 = jnp.argmax(masked, axis=1)
    m2 = jnp.max(masked, axis=1)

    scores_ref[...] = scores
    wts_ref[...] = jnp.stack([m1, m2], axis=1)
    idx_ref[...] = jnp.stack([i1, i2], axis=1).astype(jnp.int32)


def kernel(x, W):
    x2d = x.reshape(-1, x.shape[-1])
    n_tokens = x2d.shape[0]
    grid = (n_tokens // TILE,)
    scores, wts, idx = pl.pallas_call(
        _router_kernel,
        grid=grid,
        in_specs=[
            pl.BlockSpec((TILE, HIDDEN_DIM), lambda i: (i, 0)),
            pl.BlockSpec((N_EXPERTS, HIDDEN_DIM), lambda i: (0, 0)),
        ],
        out_specs=[
            pl.BlockSpec((TILE, N_EXPERTS), lambda i: (i, 0)),
            pl.BlockSpec((TILE, K_TOP), lambda i: (i, 0)),
            pl.BlockSpec((TILE, K_TOP), lambda i: (i, 0)),
        ],
        out_shape=[
            jax.ShapeDtypeStruct((n_tokens, N_EXPERTS), jnp.float32),
            jax.ShapeDtypeStruct((n_tokens, K_TOP), jnp.float32),
            jax.ShapeDtypeStruct((n_tokens, K_TOP), jnp.int32),
        ],
    )(x2d, W)
    return scores, wts, idx, jnp.float32(0.0)
